# baseline (speedup 1.0000x reference)
import jax
import jax.numpy as jnp
from jax import lax
from jax.experimental import pallas as pl
from jax.experimental.pallas import tpu as pltpu
from jax.experimental.pallas import tpu_sc as plsc

_N = 10000
_D = 128
_E = 320000
_OUT = 40

_NC = 2
_NS = 16
_NW = _NC * _NS
_CHUNK = 64
_NBUF = 5
_NIDX = 10
_LG = 3
_LS = 2
_LI = 5
_GRP = _NIDX
_NCH = 320
_EPAD = _NS * _NCH * _CHUNK
_NPAD = ((_N + _NS * 8 - 1) // (_NS * 8)) * (_NS * 8)
_RPT = _NPAD // _NS


def _sc_segsum_body(x_hbm, src_hbm, dst_hbm, out_hbm,
                    sidx, didx, rows, acc, gsems, ssems, isems):
    c = lax.axis_index("c")
    s = lax.axis_index("s")

    def pipeline(nchunk, base):
        def idx_start(i, q):
            pltpu.async_copy(src_hbm.at[base + i], sidx.at[q], isems[q])
            pltpu.async_copy(dst_hbm.at[base + i], didx.at[q], isems[q])

        def idx_wait(i, q):
            pltpu.make_async_copy(src_hbm.at[base + i], sidx.at[q],
                                  isems[q]).wait()
            pltpu.make_async_copy(dst_hbm.at[base + i], didx.at[q],
                                  isems[q]).wait()

        def gather_start(i, q, r):
            pltpu.async_copy(x_hbm.at[sidx.at[q]], rows.at[r], gsems[r])

        def gather_wait(i, q, r):
            pltpu.make_async_copy(x_hbm.at[sidx.at[q]], rows.at[r],
                                  gsems[r]).wait()

        def scatter_start(i, q, r):
            pltpu.async_copy(rows.at[r], acc.at[didx.at[q]], ssems[r],
                             add=True)

        def scatter_wait(i, q, r):
            pltpu.make_async_copy(rows.at[r], acc.at[didx.at[q]],
                                  ssems[r]).wait()

        def emit_chunk(i, u, first, last):
            if (not first) or u >= _LS:
                scatter_wait(i - _LS, (u - _LS) % _NIDX, (u - _LS) % _NBUF)
            if (not last) or u < _GRP - _LI:
                idx_start(i + _LI, (u + _LI) % _NIDX)
            if (not last) or u < _GRP - _LG:
                idx_wait(i + _LG, (u + _LG) % _NIDX)
                gather_start(i + _LG, (u + _LG) % _NIDX, (u + _LG) % _NBUF)
            gather_wait(i, u % _NIDX, u % _NBUF)
            scatter_start(i, u % _NIDX, u % _NBUF)

        for i in range(_LI):
            idx_start(i, i)
        for i in range(_LG):
            idx_wait(i, i)
            gather_start(i, i, i)

        for u in range(_GRP):
            emit_chunk(u, u, True, False)

        def group(g, carry):
            b = g * _GRP
            for u in range(_GRP):
                emit_chunk(b + u, u, False, False)
            return carry

        lax.fori_loop(1, nchunk // _GRP - 1, group, 0)

        b = nchunk - _GRP
        for u in range(_GRP):
            emit_chunk(b + u, u, False, True)
        for i in range(nchunk - _LS, nchunk):
            scatter_wait(i, i % _NIDX, i % _NBUF)

    @pl.when(c == 0)
    def _():
        def zrow(t, carry):
            rows[0, t // 8, pl.ds((t % 8) * 16, 16)] = jnp.zeros(
                (16,), jnp.float32)
            return carry

        lax.fori_loop(0, _CHUNK * 8, zrow, 0)
        nfull = _RPT // _CHUNK
        for p in range(nfull):
            pltpu.sync_copy(rows.at[0],
                            acc.at[pl.ds(s * _RPT + p * _CHUNK, _CHUNK)])
        rem = _RPT - nfull * _CHUNK
        if rem:
            pltpu.sync_copy(rows.at[0].at[pl.ds(0, rem)],
                            acc.at[pl.ds(s * _RPT + nfull * _CHUNK, rem)])
        plsc.subcore_barrier()

        pipeline(_NCH, s * _NCH)

        plsc.subcore_barrier()
        pltpu.sync_copy(acc.at[pl.ds(s * _RPT, _RPT)],
                        out_hbm.at[pl.ds(s * _RPT, _RPT)])


_SC_SEGSUM_CACHE = []


def _sc_segsum(x, src_p, dst_p):
    if not _SC_SEGSUM_CACHE:
        _SC_SEGSUM_CACHE.append(pl.kernel(
            _sc_segsum_body,
            out_type=jax.ShapeDtypeStruct((_NPAD, _D), jnp.float32),
            mesh=plsc.VectorSubcoreMesh(core_axis_name="c",
                                        subcore_axis_name="s"),
            scratch_types=[
                pltpu.VMEM((_NIDX, _CHUNK), jnp.int32),
                pltpu.VMEM((_NIDX, _CHUNK), jnp.int32),
                pltpu.VMEM((_NBUF, _CHUNK, _D), jnp.float32),
                pltpu.VMEM_SHARED((_NPAD, _D), jnp.float32),
                [pltpu.SemaphoreType.DMA] * _NBUF,
                [pltpu.SemaphoreType.DMA] * _NBUF,
                [pltpu.SemaphoreType.DMA] * _NIDX,
            ],
        ))
    return _SC_SEGSUM_CACHE[0](x, src_p, dst_p)


def _row_mask():
    return (lax.broadcasted_iota(jnp.int32, (_NPAD, 1), 0) < _N).astype(jnp.float32)


def _conv_norm(agg_ref, x, Wr, br, Wo, gnw, gnb, gna):
    t = (jnp.dot(agg_ref[...], Wr[...], preferred_element_type=jnp.float32) + br[...]
         + jnp.dot(x[...], Wo[...], preferred_element_type=jnp.float32))
    mask = _row_mask()
    mean = jnp.sum(t * mask, axis=0, keepdims=True) * (1.0 / _N)
    xc = t - gna[...] * mean
    xcm = xc * mask
    var = jnp.sum(xcm * xcm, axis=0, keepdims=True) * (1.0 / _N)
    return gnw[...] * xc * lax.rsqrt(var + 1e-5) + gnb[...]


def _tc_mid_body(agg_ref, x_ref, Wr, br, Wo, gnw, gnb, gna, o_ref):
    y = _conv_norm(agg_ref, x_ref, Wr, br, Wo, gnw, gnb, gna)
    o_ref[...] = jnp.where(y >= 0, y, 0.1 * y)


def _tc_fin_body(agg_ref, x_ref, feat_ref, Wr, br, Wo, gnw, gnb, gna,
                 Wlt, bl, o_ref):
    y = _conv_norm(agg_ref, x_ref, Wr, br, Wo, gnw, gnb, gna)
    z = jnp.maximum(feat_ref[...] + y, 0.0)
    pooled = jnp.sum(z * _row_mask(), axis=0, keepdims=True) * (1.0 / _N)
    out = jnp.dot(pooled, Wlt[...], preferred_element_type=jnp.float32) + bl[...]
    o_ref[...] = jnp.maximum(out, 0.0)


_tc_mid = pl.pallas_call(
    _tc_mid_body,
    out_shape=jax.ShapeDtypeStruct((_NPAD, _D), jnp.float32),
)

_tc_fin = pl.pallas_call(
    _tc_fin_body,
    out_shape=jax.ShapeDtypeStruct((1, _D), jnp.float32),
)


def kernel(edge_index, feat,
           W_rel0, b_rel0, W_root0, gn_w0, gn_b0, gn_a0,
           W_rel1, b_rel1, W_root1, gn_w1, gn_b1, gn_a1,
           W_rel2, b_rel2, W_root2, gn_w2, gn_b2, gn_a2,
           W_rel3, b_rel3, W_root3, gn_w3, gn_b3, gn_a3,
           W_lin, b_lin):
    src = edge_index[0].astype(jnp.int32)
    dst = edge_index[1].astype(jnp.int32)
    pad = _EPAD - _E
    src_p = jnp.concatenate(
        [src, jnp.zeros((pad,), jnp.int32)]).reshape(_EPAD // _CHUNK, _CHUNK)
    dst_p = jnp.concatenate(
        [dst, jnp.full((pad,), _N, jnp.int32)]).reshape(_EPAD // _CHUNK, _CHUNK)
    feat_p = jnp.concatenate(
        [feat, jnp.zeros((_NPAD - _N, _D), jnp.float32)], axis=0)

    convs = [(W_rel0, b_rel0, W_root0), (W_rel1, b_rel1, W_root1),
             (W_rel2, b_rel2, W_root2), (W_rel3, b_rel3, W_root3)]
    norms = [(gn_w0, gn_b0, gn_a0), (gn_w1, gn_b1, gn_a1),
             (gn_w2, gn_b2, gn_a2), (gn_w3, gn_b3, gn_a3)]

    x = feat_p
    for i in range(3):
        Wr, br, Wo = convs[i]
        w, b, a = norms[i]
        part = _sc_segsum(x, src_p, dst_p)
        x = _tc_mid(part, x, Wr, br.reshape(1, _D), Wo,
                    w.reshape(1, _D), b.reshape(1, _D), a.reshape(1, _D))

    Wr, br, Wo = convs[3]
    w, b, a = norms[3]
    part = _sc_segsum(x, src_p, dst_p)
    Wlt = jnp.zeros((_D, _D), jnp.float32).at[:, :_OUT].set(W_lin.T)
    blp = jnp.zeros((1, _D), jnp.float32).at[0, :_OUT].set(b_lin)
    out = _tc_fin(part, x, feat_p, Wr, br.reshape(1, _D), Wo,
                  w.reshape(1, _D), b.reshape(1, _D), a.reshape(1, _D),
                  Wlt, blp)
    return out[0, :_OUT]


# speedup vs baseline: 1.2663x; 1.2663x over previous
import jax
import jax.numpy as jnp
from jax import lax
from jax.experimental import pallas as pl
from jax.experimental.pallas import tpu as pltpu
from jax.experimental.pallas import tpu_sc as plsc

_N = 10000
_D = 128
_E = 320000
_OUT = 40

_NC = 2
_NS = 16
_NW = _NC * _NS
_CHUNK = 64
_NBUF = 5
_NIDX = 10
_LG = 3
_LS = 2
_LI = 5
_GRP = _NIDX
_NCH = (160, 160)
_EPAD = _NS * (_NCH[0] + _NCH[1]) * _CHUNK
_NPAD = ((_N + _NS * 8 - 1) // (_NS * 8)) * (_NS * 8)
_RPT = _NPAD // _NS


def _sc_segsum_body(x_hbm, src_hbm, dst_hbm, out_hbm,
                    sidx, didx, rows, acc, gsems, ssems, isems):
    c = lax.axis_index("c")
    s = lax.axis_index("s")

    def pipeline(nchunk, base):
        def idx_start(i, q):
            pltpu.async_copy(src_hbm.at[base + i], sidx.at[q], isems[q])
            pltpu.async_copy(dst_hbm.at[base + i], didx.at[q], isems[q])

        def idx_wait(i, q):
            pltpu.make_async_copy(src_hbm.at[base + i], sidx.at[q],
                                  isems[q]).wait()
            pltpu.make_async_copy(dst_hbm.at[base + i], didx.at[q],
                                  isems[q]).wait()

        def gather_start(i, q, r):
            pltpu.async_copy(x_hbm.at[sidx.at[q]], rows.at[r], gsems[r])

        def gather_wait(i, q, r):
            pltpu.make_async_copy(x_hbm.at[sidx.at[q]], rows.at[r],
                                  gsems[r]).wait()

        def scatter_start(i, q, r):
            pltpu.async_copy(rows.at[r], acc.at[didx.at[q]], ssems[r],
                             add=True)

        def scatter_wait(i, q, r):
            pltpu.make_async_copy(rows.at[r], acc.at[didx.at[q]],
                                  ssems[r]).wait()

        def emit_chunk(i, u, first, last):
            if (not first) or u >= _LS:
                scatter_wait(i - _LS, (u - _LS) % _NIDX, (u - _LS) % _NBUF)
            if (not last) or u < _GRP - _LI:
                idx_start(i + _LI, (u + _LI) % _NIDX)
            if (not last) or u < _GRP - _LG:
                idx_wait(i + _LG, (u + _LG) % _NIDX)
                gather_start(i + _LG, (u + _LG) % _NIDX, (u + _LG) % _NBUF)
            gather_wait(i, u % _NIDX, u % _NBUF)
            scatter_start(i, u % _NIDX, u % _NBUF)

        for i in range(_LI):
            idx_start(i, i)
        for i in range(_LG):
            idx_wait(i, i)
            gather_start(i, i, i)

        for u in range(_GRP):
            emit_chunk(u, u, True, False)

        def group(g, carry):
            b = g * _GRP
            for u in range(_GRP):
                emit_chunk(b + u, u, False, False)
            return carry

        lax.fori_loop(1, nchunk // _GRP - 1, group, 0)

        b = nchunk - _GRP
        for u in range(_GRP):
            emit_chunk(b + u, u, False, True)
        for i in range(nchunk - _LS, nchunk):
            scatter_wait(i, i % _NIDX, i % _NBUF)

    def zrow(t, carry):
        rows[0, t // 8, pl.ds((t % 8) * 16, 16)] = jnp.zeros(
            (16,), jnp.float32)
        return carry

    lax.fori_loop(0, _CHUNK * 8, zrow, 0)
    nfull = _RPT // _CHUNK
    for p in range(nfull):
        pltpu.sync_copy(rows.at[0],
                        acc.at[pl.ds(s * _RPT + p * _CHUNK, _CHUNK)])
    rem = _RPT - nfull * _CHUNK
    if rem:
        pltpu.sync_copy(rows.at[0].at[pl.ds(0, rem)],
                        acc.at[pl.ds(s * _RPT + nfull * _CHUNK, rem)])
    plsc.subcore_barrier()

    @pl.when(c == 0)
    def _():
        pipeline(_NCH[0], s * _NCH[0])

    @pl.when(c == 1)
    def _():
        pipeline(_NCH[1], _NS * _NCH[0] + s * _NCH[1])

    plsc.subcore_barrier()
    pltpu.sync_copy(acc.at[pl.ds(s * _RPT, _RPT)],
                    out_hbm.at[c].at[pl.ds(s * _RPT, _RPT)])


_SC_SEGSUM_CACHE = []


def _sc_segsum(x, src_p, dst_p):
    if not _SC_SEGSUM_CACHE:
        _SC_SEGSUM_CACHE.append(pl.kernel(
            _sc_segsum_body,
            out_type=jax.ShapeDtypeStruct((_NC, _NPAD, _D), jnp.float32),
            mesh=plsc.VectorSubcoreMesh(core_axis_name="c",
                                        subcore_axis_name="s"),
            scratch_types=[
                pltpu.VMEM((_NIDX, _CHUNK), jnp.int32),
                pltpu.VMEM((_NIDX, _CHUNK), jnp.int32),
                pltpu.VMEM((_NBUF, _CHUNK, _D), jnp.float32),
                pltpu.VMEM_SHARED((_NPAD, _D), jnp.float32),
                [pltpu.SemaphoreType.DMA] * _NBUF,
                [pltpu.SemaphoreType.DMA] * _NBUF,
                [pltpu.SemaphoreType.DMA] * _NIDX,
            ],
        ))
    return _SC_SEGSUM_CACHE[0](x, src_p, dst_p)


def _row_mask():
    return (lax.broadcasted_iota(jnp.int32, (_NPAD, 1), 0) < _N).astype(jnp.float32)


def _conv_norm(agg_ref, x, Wr, br, Wo, gnw, gnb, gna):
    agg = agg_ref[0] + agg_ref[1]
    t = (jnp.dot(agg, Wr[...], preferred_element_type=jnp.float32) + br[...]
         + jnp.dot(x[...], Wo[...], preferred_element_type=jnp.float32))
    mask = _row_mask()
    mean = jnp.sum(t * mask, axis=0, keepdims=True) * (1.0 / _N)
    xc = t - gna[...] * mean
    xcm = xc * mask
    var = jnp.sum(xcm * xcm, axis=0, keepdims=True) * (1.0 / _N)
    return gnw[...] * xc * lax.rsqrt(var + 1e-5) + gnb[...]


def _tc_mid_body(agg_ref, x_ref, Wr, br, Wo, gnw, gnb, gna, o_ref):
    y = _conv_norm(agg_ref, x_ref, Wr, br, Wo, gnw, gnb, gna)
    o_ref[...] = jnp.where(y >= 0, y, 0.1 * y)


def _tc_fin_body(agg_ref, x_ref, feat_ref, Wr, br, Wo, gnw, gnb, gna,
                 Wlt, bl, o_ref):
    y = _conv_norm(agg_ref, x_ref, Wr, br, Wo, gnw, gnb, gna)
    z = jnp.maximum(feat_ref[...] + y, 0.0)
    pooled = jnp.sum(z * _row_mask(), axis=0, keepdims=True) * (1.0 / _N)
    out = jnp.dot(pooled, Wlt[...], preferred_element_type=jnp.float32) + bl[...]
    o_ref[...] = jnp.maximum(out, 0.0)


_tc_mid = pl.pallas_call(
    _tc_mid_body,
    out_shape=jax.ShapeDtypeStruct((_NPAD, _D), jnp.float32),
)

_tc_fin = pl.pallas_call(
    _tc_fin_body,
    out_shape=jax.ShapeDtypeStruct((1, _D), jnp.float32),
)


def kernel(edge_index, feat,
           W_rel0, b_rel0, W_root0, gn_w0, gn_b0, gn_a0,
           W_rel1, b_rel1, W_root1, gn_w1, gn_b1, gn_a1,
           W_rel2, b_rel2, W_root2, gn_w2, gn_b2, gn_a2,
           W_rel3, b_rel3, W_root3, gn_w3, gn_b3, gn_a3,
           W_lin, b_lin):
    src = edge_index[0].astype(jnp.int32)
    dst = edge_index[1].astype(jnp.int32)
    pad = _EPAD - _E
    src_p = jnp.concatenate(
        [src, jnp.zeros((pad,), jnp.int32)]).reshape(_EPAD // _CHUNK, _CHUNK)
    dst_p = jnp.concatenate(
        [dst, jnp.full((pad,), _N, jnp.int32)]).reshape(_EPAD // _CHUNK, _CHUNK)
    feat_p = jnp.concatenate(
        [feat, jnp.zeros((_NPAD - _N, _D), jnp.float32)], axis=0)

    convs = [(W_rel0, b_rel0, W_root0), (W_rel1, b_rel1, W_root1),
             (W_rel2, b_rel2, W_root2), (W_rel3, b_rel3, W_root3)]
    norms = [(gn_w0, gn_b0, gn_a0), (gn_w1, gn_b1, gn_a1),
             (gn_w2, gn_b2, gn_a2), (gn_w3, gn_b3, gn_a3)]

    x = feat_p
    for i in range(3):
        Wr, br, Wo = convs[i]
        w, b, a = norms[i]
        part = _sc_segsum(x, src_p, dst_p)
        x = _tc_mid(part, x, Wr, br.reshape(1, _D), Wo,
                    w.reshape(1, _D), b.reshape(1, _D), a.reshape(1, _D))

    Wr, br, Wo = convs[3]
    w, b, a = norms[3]
    part = _sc_segsum(x, src_p, dst_p)
    Wlt = jnp.zeros((_D, _D), jnp.float32).at[:, :_OUT].set(W_lin.T)
    blp = jnp.zeros((1, _D), jnp.float32).at[0, :_OUT].set(b_lin)
    out = _tc_fin(part, x, feat_p, Wr, br.reshape(1, _D), Wo,
                  w.reshape(1, _D), b.reshape(1, _D), a.reshape(1, _D),
                  Wlt, blp)
    return out[0, :_OUT]
